# baseline (speedup 1.0000x reference)
import jax
import jax.numpy as jnp
from jax import lax
from jax.experimental import pallas as pl
from jax.experimental.pallas import tpu as pltpu
from jax.experimental.pallas import tpu_sc as plsc

B, S, D = 16, 2048, 1024
L = 16
CHUNK = 128
NCHUNK = D // CHUNK
HALF = NCHUNK // 2


def _body(emb_hbm, mask_hbm, out_hbm, mask_v, idx_v, d_v, sem):
    w = lax.axis_index("s") * 2 + lax.axis_index("c")
    b = w // 2
    h = w % 2

    pltpu.sync_copy(mask_hbm, mask_v)
    iota = lax.iota(jnp.int32, L)
    m0b = mask_v[pl.ds(pl.multiple_of(b * 8, 8), L)][0]
    m1b = mask_v[pl.ds(pl.multiple_of(8 * B + b * 8, 8), L)][0]

    ms = jnp.where(m0b != 0, 0, 1)
    me = jnp.where(m1b != 0, 1, ms)

    t = jnp.where(iota < HALF, ms, me)
    idx_v[...] = b * (S * NCHUNK) + t * NCHUNK + h * HALF + (iota & (HALF - 1))

    pltpu.async_copy(emb_hbm.at[idx_v.at[pl.ds(0, 2 * HALF)]], d_v, sem).wait()

    for j in range(HALF):
        for k in range(0, CHUNK, L):
            d_v[j, pl.ds(k, L)] = (
                d_v[j, pl.ds(k, L)] + d_v[j + HALF, pl.ds(k, L)]) * 0.5

    pltpu.sync_copy(d_v.at[pl.ds(0, HALF)], out_hbm.at[w])


def kernel(sequence_embeddings, special_tokens_mask):
    emb = sequence_embeddings.reshape(B * S * NCHUNK, CHUNK)
    mask = jnp.pad(jnp.repeat(special_tokens_mask.T.reshape(-1), 8), (0, 16))
    mesh = plsc.VectorSubcoreMesh(core_axis_name="c", subcore_axis_name="s")
    out = pl.kernel(
        _body,
        out_type=jax.ShapeDtypeStruct((2 * B, HALF, CHUNK), jnp.float32),
        mesh=mesh,
        scratch_types=[
            pltpu.VMEM((2 * B * 8 + 16,), jnp.int32),
            pltpu.VMEM((L,), jnp.int32),
            pltpu.VMEM((2 * HALF, CHUNK), jnp.float32),
            pltpu.SemaphoreType.DMA,
        ],
    )(emb, mask)
    return out.reshape(B, D)


# speedup vs baseline: 7.2243x; 7.2243x over previous
import jax
import jax.numpy as jnp
from jax import lax
from jax.experimental import pallas as pl
from jax.experimental.pallas import tpu as pltpu
from jax.experimental.pallas import tpu_sc as plsc

B, S, D = 16, 2048, 1024
L = 16
HALF = D // 2


def _body(emb_hbm, mask_hbm, out_hbm, mask_v, d0_v, d1_v, sem0, sem1):
    w = lax.axis_index("s") * 2 + lax.axis_index("c")
    b = w // 2
    h = w % 2

    pltpu.sync_copy(mask_hbm, mask_v)
    m0b = mask_v[pl.ds(pl.multiple_of(b * 8, 8), L)][0]
    m1b = mask_v[pl.ds(pl.multiple_of(8 * B + b * 8, 8), L)][0]

    ms = jnp.where(m0b != 0, 0, 1)
    me = jnp.where(m1b != 0, 1, ms)

    c0 = h * HALF
    cp0 = pltpu.async_copy(emb_hbm.at[b * S + ms, pl.ds(c0, HALF)], d0_v, sem0)
    cp1 = pltpu.async_copy(emb_hbm.at[b * S + me, pl.ds(c0, HALF)], d1_v, sem1)
    cp0.wait()
    cp1.wait()

    for k in range(0, HALF, L):
        d0_v[pl.ds(k, L)] = (d0_v[pl.ds(k, L)] + d1_v[pl.ds(k, L)]) * 0.5

    pltpu.sync_copy(d0_v, out_hbm.at[b, pl.ds(c0, HALF)])


def kernel(sequence_embeddings, special_tokens_mask):
    emb = sequence_embeddings.reshape(B * S, D)
    mask = jnp.pad(jnp.repeat(special_tokens_mask.T.reshape(-1), 8), (0, 16))
    mesh = plsc.VectorSubcoreMesh(core_axis_name="c", subcore_axis_name="s")
    return pl.kernel(
        _body,
        out_type=jax.ShapeDtypeStruct((B, D), jnp.float32),
        mesh=mesh,
        scratch_types=[
            pltpu.VMEM((2 * B * 8 + 16,), jnp.int32),
            pltpu.VMEM((HALF,), jnp.float32),
            pltpu.VMEM((HALF,), jnp.float32),
            pltpu.SemaphoreType.DMA,
            pltpu.SemaphoreType.DMA,
        ],
    )(emb, mask)


# speedup vs baseline: 7.5989x; 1.0518x over previous
import jax
import jax.numpy as jnp
from jax import lax
from jax.experimental import pallas as pl
from jax.experimental.pallas import tpu as pltpu
from jax.experimental.pallas import tpu_sc as plsc

B, S, D = 16, 2048, 1024
L = 16
HALF = D // 2


def _body(emb_hbm, mask_hbm, out_hbm, mask_v, d_v, sem0, sem1):
    w = lax.axis_index("s") * 2 + lax.axis_index("c")
    b = w // 2
    h = w % 2
    c0 = h * HALF

    cpe = pltpu.async_copy(
        emb_hbm.at[b, pl.ds(0, 2), pl.ds(c0, HALF)], d_v, sem1)
    cpm = pltpu.async_copy(
        mask_hbm.at[pl.ds(pl.multiple_of(b * L, 8), L)], mask_v, sem0)
    cpm.wait()
    cpe.wait()

    mvec = mask_v[pl.ds(0, L)]
    ms = jnp.where(mvec[0] != 0, 0, 1)
    me = jnp.where(mvec[L // 2] != 0, 1, ms)

    @pl.when(jnp.logical_or(ms != 0, me != 1))
    def _():
        f0 = pltpu.async_copy(
            emb_hbm.at[b, ms, pl.ds(c0, HALF)], d_v.at[0], sem0)
        f1 = pltpu.async_copy(
            emb_hbm.at[b, me, pl.ds(c0, HALF)], d_v.at[1], sem1)
        f0.wait()
        f1.wait()

    for k in range(0, HALF, L):
        d_v[0, pl.ds(k, L)] = (
            d_v[0, pl.ds(k, L)] + d_v[1, pl.ds(k, L)]) * 0.5

    pltpu.sync_copy(d_v.at[0], out_hbm.at[b, pl.ds(c0, HALF)])


def kernel(sequence_embeddings, special_tokens_mask):
    mask_rep = jnp.repeat(special_tokens_mask.reshape(-1), 8)
    mesh = plsc.VectorSubcoreMesh(core_axis_name="c", subcore_axis_name="s")
    return pl.kernel(
        _body,
        out_type=jax.ShapeDtypeStruct((B, D), jnp.float32),
        mesh=mesh,
        scratch_types=[
            pltpu.VMEM((L,), jnp.int32),
            pltpu.VMEM((2, HALF), jnp.float32),
            pltpu.SemaphoreType.DMA,
            pltpu.SemaphoreType.DMA,
        ],
    )(sequence_embeddings, mask_rep)
